# baseline (speedup 1.0000x reference)
import functools

import jax
import jax.numpy as jnp
from jax import lax
from jax.experimental import pallas as pl
from jax.experimental.pallas import tpu as pltpu
from jax.experimental.pallas import tpu_sc as plsc

IN_DIM = 768
NUM_EXPERTS = 64
NUM_TOKENS = 32768
BLOCK_ROWS = 4096

NUM_WORKERS = 32
TOK_PER_W = NUM_TOKENS // NUM_WORKERS
CHUNK = 128
NCHUNK = TOK_PER_W // CHUNK
NEG_INF = float("-inf")


def _logits_body(x_ref, w_ref, logits_ref, z_ref):
    i = pl.program_id(0)
    logits = jnp.dot(x_ref[...], w_ref[...], preferred_element_type=jnp.float32)
    logits_ref[...] = logits
    zpart = jnp.sum(jnp.log(jnp.sum(jnp.exp(logits), axis=1)))

    @pl.when(i == 0)
    def _init():
        z_ref[...] = jnp.zeros_like(z_ref)

    z_ref[...] += zpart


def _logits_call(x_flat, w_gate):
    grid = NUM_TOKENS // BLOCK_ROWS
    return pl.pallas_call(
        _logits_body,
        grid=(grid,),
        in_specs=[
            pl.BlockSpec((BLOCK_ROWS, IN_DIM), lambda i: (i, 0)),
            pl.BlockSpec((IN_DIM, NUM_EXPERTS), lambda i: (0, 0)),
        ],
        out_specs=[
            pl.BlockSpec((BLOCK_ROWS, NUM_EXPERTS), lambda i: (i, 0)),
            pl.BlockSpec((1, 1), lambda i: (0, 0)),
        ],
        out_shape=[
            jax.ShapeDtypeStruct((NUM_TOKENS, NUM_EXPERTS), jnp.float32),
            jax.ShapeDtypeStruct((1, 1), jnp.float32),
        ],
    )(x_flat, w_gate)


_SC_MESH = plsc.VectorSubcoreMesh(core_axis_name="c", subcore_axis_name="s")


@functools.partial(
    pl.kernel,
    mesh=_SC_MESH,
    out_type=[
        jax.ShapeDtypeStruct((NUM_TOKENS, NUM_EXPERTS), jnp.float32),
        jax.ShapeDtypeStruct((NUM_WORKERS, NUM_EXPERTS), jnp.float32),
        jax.ShapeDtypeStruct((NUM_WORKERS, NUM_EXPERTS), jnp.float32),
    ],
    scratch_types=[
        pltpu.VMEM((CHUNK, NUM_EXPERTS), jnp.float32),
        pltpu.VMEM((CHUNK, NUM_EXPERTS), jnp.float32),
        pltpu.VMEM((16, NUM_EXPERTS), jnp.float32),
        pltpu.VMEM((16, NUM_EXPERTS), jnp.float32),
        pltpu.VMEM((NUM_EXPERTS,), jnp.float32),
    ],
    compiler_params=pltpu.CompilerParams(needs_layout_passes=False),
)
def _route_sc(logits_hbm, gates_hbm, pimp_hbm, pload_hbm,
              lbuf, gbuf, aimp, aload, rowbuf):
    wid = lax.axis_index("s") * 2 + lax.axis_index("c")
    lanes = lax.iota(jnp.int32, 16)
    zeros16 = jnp.zeros((16,), jnp.float32)
    ones16 = jnp.ones((16,), jnp.float32)

    def _zero_acc(k, _):
        def _zr(t, _):
            aimp[t, pl.ds(k * 16, 16)] = zeros16
            aload[t, pl.ds(k * 16, 16)] = zeros16
            return 0

        lax.fori_loop(0, 16, _zr, 0)
        return 0

    lax.fori_loop(0, NUM_EXPERTS // 16, _zero_acc, 0)

    for c in range(NCHUNK):
        tok_off = wid * TOK_PER_W + c * CHUNK
        pltpu.sync_copy(logits_hbm.at[pl.ds(tok_off, CHUNK)], lbuf)

        def _zero_g(t, _):
            for k in range(NUM_EXPERTS // 16):
                gbuf[t, pl.ds(k * 16, 16)] = zeros16
            return 0

        lax.fori_loop(0, CHUNK, _zero_g, 0)

        def _group(j, _):
            rows = j * 16 + lanes

            def _emax(e, carry):
                m1, i1, m2, i2 = carry
                ev = jnp.broadcast_to(e, (16,))
                v = plsc.load_gather(lbuf, [rows, ev])
                c1 = v > m1
                c2 = v > m2
                m2n = jnp.where(c1, m1, jnp.where(c2, v, m2))
                i2n = jnp.where(c1, i1, jnp.where(c2, ev, i2))
                m1n = jnp.where(c1, v, m1)
                i1n = jnp.where(c1, ev, i1)
                return m1n, i1n, m2n, i2n

            init = (jnp.full((16,), NEG_INF, jnp.float32),
                    jnp.zeros((16,), jnp.int32),
                    jnp.full((16,), NEG_INF, jnp.float32),
                    jnp.zeros((16,), jnp.int32))
            m1, i1, m2, i2 = lax.fori_loop(0, NUM_EXPERTS, _emax, init)

            t = jnp.exp(m2 - m1)
            denom = t + 1.0
            g1 = 1.0 / denom
            g2 = t / denom
            plsc.store_scatter(gbuf, [rows, i1], g1)
            plsc.store_scatter(gbuf, [rows, i2], g2)
            plsc.addupdate_scatter(aimp, [lanes, i1], g1)
            plsc.addupdate_scatter(aimp, [lanes, i2], g2)
            plsc.addupdate_scatter(aload, [lanes, i1], ones16)
            l2 = jnp.where(g2 > 0.0, 1.0, 0.0)
            plsc.addupdate_scatter(aload, [lanes, i2], l2)
            return 0

        lax.fori_loop(0, CHUNK // 16, _group, 0)

        pltpu.sync_copy(gbuf, gates_hbm.at[pl.ds(tok_off, CHUNK)])

    for acc, out_hbm in ((aimp, pimp_hbm), (aload, pload_hbm)):
        for k in range(NUM_EXPERTS // 16):
            def _lanesum(t, s, acc=acc, k=k):
                return s + acc[t, pl.ds(k * 16, 16)]

            rowbuf[pl.ds(k * 16, 16)] = lax.fori_loop(0, 16, _lanesum, zeros16)
        pltpu.sync_copy(rowbuf, out_hbm.at[wid])


def _finalize_body(pimp_ref, pload_ref, z_ref, imp_ref, load_ref, lb_ref):
    imp = jnp.sum(pimp_ref[...], axis=0, keepdims=True)
    loadf = jnp.sum(pload_ref[...], axis=0, keepdims=True)
    imp_ref[...] = imp
    load_ref[...] = loadf.astype(jnp.int32)

    def cv_sq(v):
        mean = jnp.mean(v)
        var = jnp.sum((v - mean) ** 2) / (v.size - 1)
        return var / (mean * mean + 1e-10)

    zl = z_ref[0, 0] / NUM_TOKENS
    lb_ref[...] = (cv_sq(imp[0, :]) + cv_sq(loadf[0, :]) + zl).reshape(1, 1)


def _finalize_call(pimp, pload, zacc):
    return pl.pallas_call(
        _finalize_body,
        out_shape=[
            jax.ShapeDtypeStruct((1, NUM_EXPERTS), jnp.float32),
            jax.ShapeDtypeStruct((1, NUM_EXPERTS), jnp.int32),
            jax.ShapeDtypeStruct((1, 1), jnp.float32),
        ],
    )(pimp, pload, zacc)


def kernel(x_flat, w_gate, w_noise):
    del w_noise
    logits, zacc = _logits_call(x_flat, w_gate)
    gates, pimp, pload = _route_sc(logits)
    imp, load, lb = _finalize_call(pimp, pload, zacc)
    return (gates, load.reshape(NUM_EXPERTS), logits, lb.reshape(()),
            imp.reshape(NUM_EXPERTS))


# speedup vs baseline: 1.1388x; 1.1388x over previous
import functools

import jax
import jax.numpy as jnp
from jax import lax
from jax.experimental import pallas as pl
from jax.experimental.pallas import tpu as pltpu
from jax.experimental.pallas import tpu_sc as plsc

IN_DIM = 768
NUM_EXPERTS = 64
NUM_TOKENS = 32768
BLOCK_ROWS = 4096

NUM_WORKERS = 32
TOK_PER_W = NUM_TOKENS // NUM_WORKERS
CHUNK = 128
NCHUNK = TOK_PER_W // CHUNK
NEG_INF = float("-inf")


def _logits_body(x_ref, w_ref, logits_ref, z_ref):
    i = pl.program_id(0)
    logits = jnp.dot(x_ref[...], w_ref[...], preferred_element_type=jnp.float32)
    logits_ref[...] = logits
    zpart = jnp.sum(jnp.log(jnp.sum(jnp.exp(logits), axis=1)))

    @pl.when(i == 0)
    def _init():
        z_ref[...] = jnp.zeros_like(z_ref)

    z_ref[...] += zpart


def _logits_call(x_flat, w_gate):
    grid = NUM_TOKENS // BLOCK_ROWS
    return pl.pallas_call(
        _logits_body,
        grid=(grid,),
        in_specs=[
            pl.BlockSpec((BLOCK_ROWS, IN_DIM), lambda i: (i, 0)),
            pl.BlockSpec((IN_DIM, NUM_EXPERTS), lambda i: (0, 0)),
        ],
        out_specs=[
            pl.BlockSpec((BLOCK_ROWS, NUM_EXPERTS), lambda i: (i, 0)),
            pl.BlockSpec((1, 1), lambda i: (0, 0)),
        ],
        out_shape=[
            jax.ShapeDtypeStruct((NUM_TOKENS, NUM_EXPERTS), jnp.float32),
            jax.ShapeDtypeStruct((1, 1), jnp.float32),
        ],
    )(x_flat, w_gate)


_SC_MESH = plsc.VectorSubcoreMesh(core_axis_name="c", subcore_axis_name="s")


@functools.partial(
    pl.kernel,
    mesh=_SC_MESH,
    out_type=[
        jax.ShapeDtypeStruct((NUM_TOKENS, NUM_EXPERTS), jnp.float32),
        jax.ShapeDtypeStruct((NUM_WORKERS, NUM_EXPERTS), jnp.float32),
        jax.ShapeDtypeStruct((NUM_WORKERS, NUM_EXPERTS), jnp.float32),
    ],
    scratch_types=[
        pltpu.VMEM((CHUNK, NUM_EXPERTS), jnp.float32),
        pltpu.VMEM((CHUNK, NUM_EXPERTS), jnp.float32),
        pltpu.VMEM((16, NUM_EXPERTS), jnp.float32),
        pltpu.VMEM((16, NUM_EXPERTS), jnp.float32),
        pltpu.VMEM((NUM_EXPERTS,), jnp.float32),
    ],
    compiler_params=pltpu.CompilerParams(needs_layout_passes=False),
)
def _route_sc(logits_hbm, gates_hbm, pimp_hbm, pload_hbm,
              lbuf, gbuf, aimp, aload, rowbuf):
    wid = lax.axis_index("s") * 2 + lax.axis_index("c")
    lanes = lax.iota(jnp.int32, 16)
    zeros16 = jnp.zeros((16,), jnp.float32)
    ones16 = jnp.ones((16,), jnp.float32)

    def _merge(A, B):
        a1, ai1, a2, ai2 = A
        b1, bi1, b2, bi2 = B
        c = a1 >= b1
        ca = a2 >= b1
        cb = b2 > a1
        m1 = jnp.where(c, a1, b1)
        i1 = jnp.where(c, ai1, bi1)
        m2 = jnp.where(c, jnp.where(ca, a2, b1), jnp.where(cb, b2, a1))
        i2 = jnp.where(c, jnp.where(ca, ai2, bi1), jnp.where(cb, bi2, ai1))
        return m1, i1, m2, i2

    def _zero_acc(k, _):
        def _zr(t, _):
            aimp[t, pl.ds(k * 16, 16)] = zeros16
            aload[t, pl.ds(k * 16, 16)] = zeros16
            return 0

        lax.fori_loop(0, 16, _zr, 0)
        return 0

    lax.fori_loop(0, NUM_EXPERTS // 16, _zero_acc, 0)

    idx_const = [jnp.full((16,), e, jnp.int32) for e in range(NUM_EXPERTS)]

    def _chunk(c, _):
        tok_off = wid * TOK_PER_W + c * CHUNK
        pltpu.sync_copy(logits_hbm.at[pl.ds(tok_off, CHUNK)], lbuf)

        def _zero_g(t, _):
            for k in range(NUM_EXPERTS // 16):
                gbuf[t, pl.ds(k * 16, 16)] = zeros16
            return 0

        lax.fori_loop(0, CHUNK, _zero_g, 0)

        def _group(j, _):
            rows = j * 16 + lanes
            run = None
            for b in range(NUM_EXPERTS // 8):
                v = [plsc.load_gather(lbuf, [rows, idx_const[8 * b + t]])
                     for t in range(8)]
                nodes = []
                for p in range(4):
                    e0, e1 = 8 * b + 2 * p, 8 * b + 2 * p + 1
                    a, bb = v[2 * p], v[2 * p + 1]
                    cc = a >= bb
                    nodes.append((jnp.where(cc, a, bb),
                                  jnp.where(cc, idx_const[e0], idx_const[e1]),
                                  jnp.where(cc, bb, a),
                                  jnp.where(cc, idx_const[e1], idx_const[e0])))
                blk = _merge(_merge(nodes[0], nodes[1]),
                             _merge(nodes[2], nodes[3]))
                run = blk if run is None else _merge(run, blk)
            m1, i1, m2, i2 = run

            t = jnp.exp(m2 - m1)
            denom = t + 1.0
            g1 = 1.0 / denom
            g2 = t / denom
            plsc.store_scatter(gbuf, [rows, i1], g1)
            plsc.store_scatter(gbuf, [rows, i2], g2)
            plsc.addupdate_scatter(aimp, [lanes, i1], g1)
            plsc.addupdate_scatter(aimp, [lanes, i2], g2)
            plsc.addupdate_scatter(aload, [lanes, i1], ones16)
            l2 = jnp.where(g2 > 0.0, 1.0, 0.0)
            plsc.addupdate_scatter(aload, [lanes, i2], l2)
            return 0

        lax.fori_loop(0, CHUNK // 16, _group, 0)

        pltpu.sync_copy(gbuf, gates_hbm.at[pl.ds(tok_off, CHUNK)])
        return 0

    lax.fori_loop(0, NCHUNK, _chunk, 0)

    for acc, out_hbm in ((aimp, pimp_hbm), (aload, pload_hbm)):
        for k in range(NUM_EXPERTS // 16):
            def _lanesum(t, s, acc=acc, k=k):
                return s + acc[t, pl.ds(k * 16, 16)]

            rowbuf[pl.ds(k * 16, 16)] = lax.fori_loop(0, 16, _lanesum, zeros16)
        pltpu.sync_copy(rowbuf, out_hbm.at[wid])


def _finalize_body(pimp_ref, pload_ref, z_ref, imp_ref, load_ref, lb_ref):
    imp = jnp.sum(pimp_ref[...], axis=0, keepdims=True)
    loadf = jnp.sum(pload_ref[...], axis=0, keepdims=True)
    imp_ref[...] = imp
    load_ref[...] = loadf.astype(jnp.int32)

    def cv_sq(v):
        mean = jnp.mean(v)
        var = jnp.sum((v - mean) ** 2) / (v.size - 1)
        return var / (mean * mean + 1e-10)

    zl = z_ref[0, 0] / NUM_TOKENS
    lb_ref[...] = (cv_sq(imp[0, :]) + cv_sq(loadf[0, :]) + zl).reshape(1, 1)


def _finalize_call(pimp, pload, zacc):
    return pl.pallas_call(
        _finalize_body,
        out_shape=[
            jax.ShapeDtypeStruct((1, NUM_EXPERTS), jnp.float32),
            jax.ShapeDtypeStruct((1, NUM_EXPERTS), jnp.int32),
            jax.ShapeDtypeStruct((1, 1), jnp.float32),
        ],
    )(pimp, pload, zacc)


def kernel(x_flat, w_gate, w_noise):
    del w_noise
    logits, zacc = _logits_call(x_flat, w_gate)
    gates, pimp, pload = _route_sc(logits)
    imp, load, lb = _finalize_call(pimp, pload, zacc)
    return (gates, load.reshape(NUM_EXPERTS), logits, lb.reshape(()),
            imp.reshape(NUM_EXPERTS))


# speedup vs baseline: 1.2163x; 1.0681x over previous
import functools

import jax
import jax.numpy as jnp
from jax import lax
from jax.experimental import pallas as pl
from jax.experimental.pallas import tpu as pltpu
from jax.experimental.pallas import tpu_sc as plsc

IN_DIM = 768
NUM_EXPERTS = 64
NUM_TOKENS = 32768
BLOCK_ROWS = 4096

NUM_WORKERS = 32
TOK_PER_W = NUM_TOKENS // NUM_WORKERS
CHUNK = 128
NCHUNK = TOK_PER_W // CHUNK
NEG_INF = float("-inf")


def _logits_body(x_ref, w_ref, logits_ref, z_ref):
    i = pl.program_id(0)
    logits = jnp.dot(x_ref[...], w_ref[...], preferred_element_type=jnp.float32)
    logits_ref[...] = logits
    zpart = jnp.sum(jnp.log(jnp.sum(jnp.exp(logits), axis=1)))

    @pl.when(i == 0)
    def _init():
        z_ref[...] = jnp.zeros_like(z_ref)

    z_ref[...] += zpart


def _logits_call(x_flat, w_gate):
    grid = NUM_TOKENS // BLOCK_ROWS
    return pl.pallas_call(
        _logits_body,
        grid=(grid,),
        in_specs=[
            pl.BlockSpec((BLOCK_ROWS, IN_DIM), lambda i: (i, 0)),
            pl.BlockSpec((IN_DIM, NUM_EXPERTS), lambda i: (0, 0)),
        ],
        out_specs=[
            pl.BlockSpec((BLOCK_ROWS, NUM_EXPERTS), lambda i: (i, 0)),
            pl.BlockSpec((1, 1), lambda i: (0, 0)),
        ],
        out_shape=[
            jax.ShapeDtypeStruct((NUM_TOKENS, NUM_EXPERTS), jnp.float32),
            jax.ShapeDtypeStruct((1, 1), jnp.float32),
        ],
    )(x_flat, w_gate)


_SC_MESH = plsc.VectorSubcoreMesh(core_axis_name="c", subcore_axis_name="s")


@functools.partial(
    pl.kernel,
    mesh=_SC_MESH,
    out_type=[
        jax.ShapeDtypeStruct((NUM_TOKENS, NUM_EXPERTS), jnp.float32),
        jax.ShapeDtypeStruct((NUM_WORKERS, NUM_EXPERTS), jnp.float32),
        jax.ShapeDtypeStruct((NUM_WORKERS, NUM_EXPERTS), jnp.float32),
    ],
    scratch_types=[
        pltpu.VMEM((2, CHUNK, NUM_EXPERTS), jnp.float32),
        pltpu.VMEM((2, CHUNK, NUM_EXPERTS), jnp.float32),
        pltpu.VMEM((16, NUM_EXPERTS), jnp.float32),
        pltpu.VMEM((16, NUM_EXPERTS), jnp.float32),
        pltpu.VMEM((NUM_EXPERTS,), jnp.float32),
        pltpu.SemaphoreType.DMA,
        pltpu.SemaphoreType.DMA,
        pltpu.SemaphoreType.DMA,
        pltpu.SemaphoreType.DMA,
    ],
    compiler_params=pltpu.CompilerParams(needs_layout_passes=False),
)
def _route_sc(logits_hbm, gates_hbm, pimp_hbm, pload_hbm,
              lbuf2, gbuf2, aimp, aload, rowbuf,
              sem_in0, sem_in1, sem_out0, sem_out1):
    wid = lax.axis_index("s") * 2 + lax.axis_index("c")
    lanes = lax.iota(jnp.int32, 16)
    zeros16 = jnp.zeros((16,), jnp.float32)
    ones16 = jnp.ones((16,), jnp.float32)

    def _merge(A, B):
        a1, ai1, a2, ai2 = A
        b1, bi1, b2, bi2 = B
        c = a1 >= b1
        ca = a2 >= b1
        cb = b2 > a1
        m1 = jnp.where(c, a1, b1)
        i1 = jnp.where(c, ai1, bi1)
        m2 = jnp.where(c, jnp.where(ca, a2, b1), jnp.where(cb, b2, a1))
        i2 = jnp.where(c, jnp.where(ca, ai2, bi1), jnp.where(cb, bi2, ai1))
        return m1, i1, m2, i2

    def _zero_acc(k, _):
        def _zr(t, _):
            aimp[t, pl.ds(k * 16, 16)] = zeros16
            aload[t, pl.ds(k * 16, 16)] = zeros16
            return 0

        lax.fori_loop(0, 16, _zr, 0)
        return 0

    lax.fori_loop(0, NUM_EXPERTS // 16, _zero_acc, 0)

    idx_const = [jnp.full((16,), e, jnp.int32) for e in range(NUM_EXPERTS)]

    def _top2(lbuf, rows):
        run = None
        for b in range(NUM_EXPERTS // 8):
            v = [plsc.load_gather(lbuf, [rows, idx_const[8 * b + t]])
                 for t in range(8)]
            nodes = []
            for p in range(4):
                e0, e1 = 8 * b + 2 * p, 8 * b + 2 * p + 1
                a, bb = v[2 * p], v[2 * p + 1]
                cc = a >= bb
                nodes.append((jnp.where(cc, a, bb),
                              jnp.where(cc, idx_const[e0], idx_const[e1]),
                              jnp.where(cc, bb, a),
                              jnp.where(cc, idx_const[e1], idx_const[e0])))
            blk = _merge(_merge(nodes[0], nodes[1]),
                         _merge(nodes[2], nodes[3]))
            run = blk if run is None else _merge(run, blk)
        return run

    def _emit(lbuf, gbuf, rows):
        m1, i1, m2, i2 = _top2(lbuf, rows)
        t = jnp.exp(m2 - m1)
        denom = t + 1.0
        g1 = 1.0 / denom
        g2 = t / denom
        plsc.store_scatter(gbuf, [rows, i1], g1)
        plsc.store_scatter(gbuf, [rows, i2], g2)
        plsc.addupdate_scatter(aimp, [lanes, i1], g1)
        plsc.addupdate_scatter(aimp, [lanes, i2], g2)
        plsc.addupdate_scatter(aload, [lanes, i1], ones16)
        l2 = jnp.where(g2 > 0.0, 1.0, 0.0)
        plsc.addupdate_scatter(aload, [lanes, i2], l2)

    tok0 = wid * TOK_PER_W
    sems_in = (sem_in0, sem_in1)
    sems_out = (sem_out0, sem_out1)
    in_flight_out = [None, None]

    first_in = pltpu.async_copy(logits_hbm.at[pl.ds(tok0, CHUNK)],
                                lbuf2.at[0], sems_in[0])
    for c in range(NCHUNK):
        buf = c % 2
        lbuf = lbuf2.at[buf]
        gbuf = gbuf2.at[buf]
        (first_in if c == 0 else in_wait).wait()
        if c + 1 < NCHUNK:
            in_wait = pltpu.async_copy(
                logits_hbm.at[pl.ds(tok0 + (c + 1) * CHUNK, CHUNK)],
                lbuf2.at[1 - buf], sems_in[(c + 1) % 2])
        if in_flight_out[buf] is not None:
            in_flight_out[buf].wait()

        def _zero_g(t, _, gbuf=gbuf):
            for k in range(NUM_EXPERTS // 16):
                gbuf[t, pl.ds(k * 16, 16)] = zeros16
            return 0

        lax.fori_loop(0, CHUNK, _zero_g, 0)

        def _group(j, _, lbuf=lbuf, gbuf=gbuf):
            _emit(lbuf, gbuf, j * 32 + lanes)
            _emit(lbuf, gbuf, j * 32 + 16 + lanes)
            return 0

        lax.fori_loop(0, CHUNK // 32, _group, 0)

        in_flight_out[buf] = pltpu.async_copy(
            gbuf, gates_hbm.at[pl.ds(tok0 + c * CHUNK, CHUNK)], sems_out[buf])

    for h in in_flight_out:
        if h is not None:
            h.wait()

    for acc, out_hbm in ((aimp, pimp_hbm), (aload, pload_hbm)):
        for k in range(NUM_EXPERTS // 16):
            def _lanesum(t, s, acc=acc, k=k):
                return s + acc[t, pl.ds(k * 16, 16)]

            rowbuf[pl.ds(k * 16, 16)] = lax.fori_loop(0, 16, _lanesum, zeros16)
        pltpu.sync_copy(rowbuf, out_hbm.at[wid])


def _finalize_body(pimp_ref, pload_ref, z_ref, imp_ref, load_ref, lb_ref):
    imp = jnp.sum(pimp_ref[...], axis=0, keepdims=True)
    loadf = jnp.sum(pload_ref[...], axis=0, keepdims=True)
    imp_ref[...] = imp
    load_ref[...] = loadf.astype(jnp.int32)

    def cv_sq(v):
        mean = jnp.mean(v)
        var = jnp.sum((v - mean) ** 2) / (v.size - 1)
        return var / (mean * mean + 1e-10)

    zl = z_ref[0, 0] / NUM_TOKENS
    lb_ref[...] = (cv_sq(imp[0, :]) + cv_sq(loadf[0, :]) + zl).reshape(1, 1)


def _finalize_call(pimp, pload, zacc):
    return pl.pallas_call(
        _finalize_body,
        out_shape=[
            jax.ShapeDtypeStruct((1, NUM_EXPERTS), jnp.float32),
            jax.ShapeDtypeStruct((1, NUM_EXPERTS), jnp.int32),
            jax.ShapeDtypeStruct((1, 1), jnp.float32),
        ],
    )(pimp, pload, zacc)


def kernel(x_flat, w_gate, w_noise):
    del w_noise
    logits, zacc = _logits_call(x_flat, w_gate)
    gates, pimp, pload = _route_sc(logits)
    imp, load, lb = _finalize_call(pimp, pload, zacc)
    return (gates, load.reshape(NUM_EXPERTS), logits, lb.reshape(()),
            imp.reshape(NUM_EXPERTS))
